# baseline (speedup 1.0000x reference)
import functools
import math

import jax
import jax.numpy as jnp
from jax import lax
from jax.experimental import pallas as pl
from jax.experimental.pallas import tpu as pltpu
from jax.experimental.pallas import tpu_sc as plsc

_ND = 50
_HD = 10
_NI = 101
_OW = 30
_L = 16

_INV_PI = float(1.0 / jnp.pi)
_PI_HI = 3.140625
_PI_LO = 9.676535897932e-4
_S = (-1.0 / 6, 1.0 / 120, -1.0 / 5040, 1.0 / 362880,
      -1.0 / 39916800, 1.0 / 6227020800)


def _sin(v):
    k = (v * _INV_PI).astype(jnp.int32)
    kf = k.astype(jnp.float32)
    r = (v - kf * _PI_HI) - kf * _PI_LO
    r = jnp.where((k & 1) == 1, -r, r)
    r2 = r * r
    p = _S[5]
    for c in (_S[4], _S[3], _S[2], _S[1], _S[0]):
        p = p * r2 + c
    return r + r * r2 * p


def _lane_sum(v):
    lane = lax.iota(jnp.int32, _L)
    for k in (8, 4, 2, 1):
        v = v + v.at[lane ^ k].get(mode="promise_in_bounds")
    return v


def _body(x_hbm, hw_hbm, w_hbm, tbl_hbm, out_hbm,
          x_v, hw_v, w_v, rows_v, o_v,
          sem_x, sem_hw, sem_w, sem_g):
    tile0 = jnp.logical_and(lax.axis_index("c") == 0, lax.axis_index("s") == 0)

    @pl.when(tile0)
    def _():
        cp_x = pltpu.make_async_copy(x_hbm, x_v.at[pl.ds(0, 1)], sem_x)
        cp_hw = pltpu.make_async_copy(hw_hbm, hw_v.at[pl.ds(0, _HD)], sem_hw)
        cp_w = pltpu.make_async_copy(w_hbm, w_v.at[pl.ds(0, _ND * _HD)], sem_w)
        cp_x.start()
        cp_hw.start()
        cp_w.start()

        cp_x.wait()
        xs = x_v[pl.ds(0, _L)][0]
        idx = jnp.clip((xs * (_NI - 1.0)).astype(jnp.int32), 0, _NI - 1)
        cp_g = pltpu.make_async_copy(tbl_hbm.at[:, idx, :], rows_v, sem_g)
        cp_g.start()

        cp_hw.wait()
        cp_w.wait()
        lane = lax.iota(jnp.int32, _L)
        hwv = hw_v[pl.ds(0, _L)]
        sims = []
        for g in range(4):
            base = lane * _HD + (g * _L * _HD)
            acc = jnp.zeros((_L,), jnp.float32)
            for kk in range(_HD):
                col = plsc.load_gather(w_v, [base + kk])
                hwk = hwv.at[jnp.full((_L,), kk, jnp.int32)].get(
                    mode="promise_in_bounds")
                acc = acc + hwk * col
            sims.append(acc * (1.0 / math.sqrt(_HD)))

        es = [jnp.exp(_sin(s)) for s in sims]
        es[3] = jnp.where(lane < (_ND - 3 * _L), es[3], 0.0)
        total = _lane_sum(es[0] + es[1] + es[2] + es[3])

        cp_g.wait()
        acc0 = jnp.zeros((_L,), jnp.float32)
        acc1 = jnp.zeros((_L,), jnp.float32)
        for i in range(_ND):
            ilane = jnp.full((_L,), i % _L, jnp.int32)
            eb = es[i // _L].at[ilane].get(mode="promise_in_bounds")
            acc0 = acc0 + eb * rows_v[i, pl.ds(0, _L)]
            acc1 = acc1 + eb * rows_v[i, pl.ds(_OW - _L, _L)]
        inv = 1.0 / total
        o_v[pl.ds(0, _L)] = acc0 * inv
        o_v[pl.ds(_OW - _L, _L)] = acc1 * inv
        pltpu.sync_copy(o_v.at[pl.ds(0, _OW)], out_hbm)


@functools.partial(jax.jit, static_argnames=())
def kernel(x, hw, hw_embed_weight, hpn_tables):
    run = pl.kernel(
        _body,
        out_type=jax.ShapeDtypeStruct((_OW,), jnp.float32),
        mesh=plsc.VectorSubcoreMesh(core_axis_name="c", subcore_axis_name="s"),
        scratch_types=[
            pltpu.VMEM((_L,), jnp.float32),
            pltpu.VMEM((_L,), jnp.float32),
            pltpu.VMEM((4 * _L * _HD,), jnp.float32),
            pltpu.VMEM((_ND, _OW), jnp.float32),
            pltpu.VMEM((2 * _L,), jnp.float32),
            pltpu.SemaphoreType.DMA,
            pltpu.SemaphoreType.DMA,
            pltpu.SemaphoreType.DMA,
            pltpu.SemaphoreType.DMA,
        ],
        compiler_params=pltpu.CompilerParams(
            needs_layout_passes=False, use_tc_tiling_on_sc=False),
    )
    out = run(x.reshape(1), hw, hw_embed_weight.reshape(_ND * _HD), hpn_tables)
    return out.reshape(6, 5)


# speedup vs baseline: 1.0172x; 1.0172x over previous
import functools
import math

import jax
import jax.numpy as jnp
from jax import lax
from jax.experimental import pallas as pl
from jax.experimental.pallas import tpu as pltpu
from jax.experimental.pallas import tpu_sc as plsc

_ND = 50
_HD = 10
_NI = 101
_OW = 30
_L = 16

_INV_PI = float(1.0 / jnp.pi)
_PI_HI = 3.140625
_PI_LO = 9.676535897932e-4
_S = (-1.0 / 6, 1.0 / 120, -1.0 / 5040, 1.0 / 362880,
      -1.0 / 39916800, 1.0 / 6227020800)


def _sin(v):
    k = (v * _INV_PI).astype(jnp.int32)
    kf = k.astype(jnp.float32)
    r = (v - kf * _PI_HI) - kf * _PI_LO
    r = jnp.where((k & 1) == 1, -r, r)
    r2 = r * r
    p = _S[5]
    for c in (_S[4], _S[3], _S[2], _S[1], _S[0]):
        p = p * r2 + c
    return r + r * r2 * p


def _lane_sum(v):
    lane = lax.iota(jnp.int32, _L)
    for k in (8, 4, 2, 1):
        v = v + v.at[lane ^ k].get(mode="promise_in_bounds")
    return v


def _body(x_hbm, hw_hbm, w_hbm, tbl_hbm, out_hbm,
          x_v, hw_v, w_v, rows_v, o_v,
          sem_x, sem_hw, sem_w, sem_g):
    tile0 = jnp.logical_and(lax.axis_index("c") == 0, lax.axis_index("s") == 0)

    @pl.when(tile0)
    def _():
        cp_x = pltpu.make_async_copy(x_hbm.at[0], x_v.at[pl.ds(0, 1)], sem_x)
        cp_hw = pltpu.make_async_copy(hw_hbm, hw_v.at[pl.ds(0, _HD)], sem_hw)
        cp_w = pltpu.make_async_copy(w_hbm, w_v, sem_w)
        cp_x.start()
        cp_hw.start()
        cp_w.start()

        cp_x.wait()
        xs = x_v[pl.ds(0, _L)][0]
        idx = jnp.clip((xs * (_NI - 1.0)).astype(jnp.int32), 0, _NI - 1)
        cp_g = pltpu.make_async_copy(tbl_hbm.at[:, idx, :], rows_v, sem_g)
        cp_g.start()

        cp_hw.wait()
        cp_w.wait()
        lane = lax.iota(jnp.int32, _L)
        hwv = hw_v[pl.ds(0, _L)]
        sims = []
        for g in range(4):
            dev = jnp.minimum(lane + (g * _L), _ND - 1)
            acc = jnp.zeros((_L,), jnp.float32)
            for kk in range(_HD):
                col = plsc.load_gather(w_v, [dev, jnp.full((_L,), kk, jnp.int32)])
                hwk = hwv.at[jnp.full((_L,), kk, jnp.int32)].get(
                    mode="promise_in_bounds")
                acc = acc + hwk * col
            sims.append(acc * (1.0 / math.sqrt(_HD)))

        es = [jnp.exp(_sin(s)) for s in sims]
        es[3] = jnp.where(lane < (_ND - 3 * _L), es[3], 0.0)
        total = _lane_sum(es[0] + es[1] + es[2] + es[3])

        cp_g.wait()
        acc0 = jnp.zeros((_L,), jnp.float32)
        acc1 = jnp.zeros((_L,), jnp.float32)
        for i in range(_ND):
            ilane = jnp.full((_L,), i % _L, jnp.int32)
            eb = es[i // _L].at[ilane].get(mode="promise_in_bounds")
            acc0 = acc0 + eb * rows_v[i, pl.ds(0, _L)]
            acc1 = acc1 + eb * rows_v[i, pl.ds(_OW - _L, _L)]
        inv = 1.0 / total
        p0 = lane
        p1 = lane + (_OW - _L)
        plsc.store_scatter(o_v, [p0 // 5, p0 % 5], acc0 * inv)
        plsc.store_scatter(o_v, [p1 // 5, p1 % 5], acc1 * inv)
        pltpu.sync_copy(o_v, out_hbm)


@functools.partial(jax.jit, static_argnames=())
def kernel(x, hw, hw_embed_weight, hpn_tables):
    run = pl.kernel(
        _body,
        out_type=jax.ShapeDtypeStruct((6, 5), jnp.float32),
        mesh=plsc.VectorSubcoreMesh(core_axis_name="c", subcore_axis_name="s",
                                    num_cores=1),
        scratch_types=[
            pltpu.VMEM((_L,), jnp.float32),
            pltpu.VMEM((_L,), jnp.float32),
            pltpu.VMEM((_ND, _HD), jnp.float32),
            pltpu.VMEM((_ND, _OW), jnp.float32),
            pltpu.VMEM((6, 5), jnp.float32),
            pltpu.SemaphoreType.DMA,
            pltpu.SemaphoreType.DMA,
            pltpu.SemaphoreType.DMA,
            pltpu.SemaphoreType.DMA,
        ],
        compiler_params=pltpu.CompilerParams(
            needs_layout_passes=False, use_tc_tiling_on_sc=False),
    )
    return run(x, hw, hw_embed_weight, hpn_tables)


# speedup vs baseline: 8.2151x; 8.0762x over previous
import functools
import math

import jax
import jax.numpy as jnp
from jax.experimental import pallas as pl
from jax.experimental.pallas import tpu as pltpu

_ND = 50
_HD = 10
_NI = 101
_OW = 30


def _body(x_sm, hw_v, wt_v, tbl_v, out_v):
    v = x_sm[0, 0] * (_NI - 1)
    t = (v + 0.5).astype(jnp.int32)
    tie_odd = jnp.logical_and(t.astype(jnp.float32) == v + 0.5, (t & 1) == 1)
    idx = jnp.clip(jnp.where(tie_odd, t - 1, t), 0, _NI - 1)
    sim = jnp.dot(hw_v[...], wt_v[...]) * (1.0 / math.sqrt(_HD))
    e = jnp.exp(jnp.sin(sim))
    p = e / jnp.sum(e)
    pw = jnp.transpose(p)
    lane = jax.lax.broadcasted_iota(jnp.int32, (1, 1, _NI), 2)
    sel = jnp.sum(jnp.where(lane == idx, tbl_v[...], 0.0), axis=2)
    red = jnp.sum(pw * sel, axis=0, keepdims=True)
    out_v[...] = jnp.concatenate(
        [red[:, 5 * r:5 * r + 5] for r in range(6)], axis=0)


@functools.partial(jax.jit, static_argnames=())
def kernel(x, hw, hw_embed_weight, hpn_tables):
    wt = hw_embed_weight.T
    tblt = jnp.transpose(hpn_tables, (0, 2, 1))
    out = pl.pallas_call(
        _body,
        out_shape=jax.ShapeDtypeStruct((6, 5), jnp.float32),
        in_specs=[
            pl.BlockSpec(memory_space=pltpu.SMEM),
            pl.BlockSpec(memory_space=pltpu.VMEM),
            pl.BlockSpec(memory_space=pltpu.VMEM),
            pl.BlockSpec(memory_space=pltpu.VMEM),
        ],
        out_specs=pl.BlockSpec(memory_space=pltpu.VMEM),
    )(x, hw.reshape(1, _HD), wt, tblt)
    return out
